# baseline (speedup 1.0000x reference)
import functools

import jax
import jax.numpy as jnp
from jax import lax
from jax.experimental import pallas as pl
from jax.experimental.pallas import tpu as pltpu
from jax.experimental.pallas import tpu_sc as plsc

N_EMBED = 1024
NUM_EXPERTS = 8
TOP_K = 2
HIDDEN = 4 * N_EMBED
N_TOKENS = 4096
N_ASSIGN = N_TOKENS * TOP_K

BLK = 256
NUM_BLOCKS = N_ASSIGN // BLK + NUM_EXPERTS
PADDED = NUM_BLOCKS * BLK

SC_CORES = 2
SC_SUBCORES = 16
NW = SC_CORES * SC_SUBCORES

G_ROWS_W = PADDED // NW
G_CH = 64
G_NCH = G_ROWS_W // G_CH

C_ROWS_W = N_TOKENS // NW
C_CH = 16
C_NCH = C_ROWS_W // C_CH

GATE_PAD = 128
GATE_ROWS = 512


def _gating_body(x_ref, gw_ref, gb_ref, i1_ref, i2_ref, wa_ref, wb_ref):
    logits = jnp.dot(x_ref[...], gw_ref[...],
                     preferred_element_type=jnp.float32) + gb_ref[...]
    col = lax.broadcasted_iota(jnp.int32, (GATE_ROWS, GATE_PAD), 1)
    m1 = jnp.max(logits, axis=1, keepdims=True)
    i1 = jnp.min(jnp.where(logits == m1, col, GATE_PAD), axis=1, keepdims=True)
    masked = jnp.where(col == i1, -jnp.inf, logits)
    m2 = jnp.max(masked, axis=1, keepdims=True)
    i2 = jnp.min(jnp.where(masked == m2, col, GATE_PAD), axis=1, keepdims=True)
    e2 = jnp.exp(m2 - m1)
    denom = 1.0 + e2
    i1_ref[...] = jnp.broadcast_to(i1, (GATE_ROWS, GATE_PAD))
    i2_ref[...] = jnp.broadcast_to(i2, (GATE_ROWS, GATE_PAD))
    wa_ref[...] = jnp.broadcast_to(1.0 / denom, (GATE_ROWS, GATE_PAD))
    wb_ref[...] = jnp.broadcast_to(e2 / denom, (GATE_ROWS, GATE_PAD))


def _gating_call(x_flat, gw_pad, gb_pad):
    n = x_flat.shape[0]
    grid = (n // GATE_ROWS,)
    out_shape = [
        jax.ShapeDtypeStruct((n, GATE_PAD), jnp.int32),
        jax.ShapeDtypeStruct((n, GATE_PAD), jnp.int32),
        jax.ShapeDtypeStruct((n, GATE_PAD), jnp.float32),
        jax.ShapeDtypeStruct((n, GATE_PAD), jnp.float32),
    ]
    spec_rows = pl.BlockSpec((GATE_ROWS, N_EMBED), lambda g: (g, 0))
    spec_out = pl.BlockSpec((GATE_ROWS, GATE_PAD), lambda g: (g, 0))
    return pl.pallas_call(
        _gating_body,
        grid=grid,
        in_specs=[
            spec_rows,
            pl.BlockSpec((N_EMBED, GATE_PAD), lambda g: (0, 0)),
            pl.BlockSpec((1, GATE_PAD), lambda g: (0, 0)),
        ],
        out_specs=[spec_out, spec_out, spec_out, spec_out],
        out_shape=out_shape,
    )(x_flat, gw_pad, gb_pad)


def _ffn1_body(be_ref, xg_ref, w1_ref, b1_ref, h_ref):
    xb = xg_ref[...]
    w1 = w1_ref[0].astype(jnp.bfloat16)
    h = jnp.dot(xb, w1, preferred_element_type=jnp.float32)
    h_ref[...] = jnp.maximum(h + b1_ref[0], 0.0).astype(jnp.bfloat16)


def _ffn1_call(be, xg, W1, b1):
    grid_spec = pltpu.PrefetchScalarGridSpec(
        num_scalar_prefetch=1,
        grid=(NUM_BLOCKS,),
        in_specs=[
            pl.BlockSpec((BLK, N_EMBED), lambda g, be: (g, 0)),
            pl.BlockSpec((1, N_EMBED, HIDDEN), lambda g, be: (be[g], 0, 0)),
            pl.BlockSpec((1, 1, HIDDEN), lambda g, be: (be[g], 0, 0)),
        ],
        out_specs=pl.BlockSpec((BLK, HIDDEN), lambda g, be: (g, 0)),
    )
    return pl.pallas_call(
        _ffn1_body,
        grid_spec=grid_spec,
        out_shape=jax.ShapeDtypeStruct((PADDED, HIDDEN), jnp.bfloat16),
    )(be, xg, W1, b1)


def _ffn2_body(be_ref, h_ref, w2_ref, b2_ref, ws_ref, out_ref):
    w2 = w2_ref[0].astype(jnp.bfloat16)
    o = jnp.dot(h_ref[...], w2, preferred_element_type=jnp.float32)
    out_ref[...] = (o + b2_ref[0]) * ws_ref[...]


def _ffn2_call(be, h, W2, b2, ws):
    grid_spec = pltpu.PrefetchScalarGridSpec(
        num_scalar_prefetch=1,
        grid=(NUM_BLOCKS,),
        in_specs=[
            pl.BlockSpec((BLK, HIDDEN), lambda g, be: (g, 0)),
            pl.BlockSpec((1, HIDDEN, N_EMBED), lambda g, be: (be[g], 0, 0)),
            pl.BlockSpec((1, 1, N_EMBED), lambda g, be: (be[g], 0, 0)),
            pl.BlockSpec((BLK, 1), lambda g, be: (g, 0)),
        ],
        out_specs=pl.BlockSpec((BLK, N_EMBED), lambda g, be: (g, 0)),
    )
    return pl.pallas_call(
        _ffn2_body,
        grid_spec=grid_spec,
        out_shape=jax.ShapeDtypeStruct((PADDED, N_EMBED), jnp.float32),
    )(be, h, W2, b2, ws)


PACKED = N_EMBED // 2


def _gather_body(x_hbm, idx_hbm, out_hbm, idx_v, rows0, rows1, gs0, gs1,
                 ws0, ws1):
    wid = lax.axis_index("s") * SC_CORES + lax.axis_index("c")
    base = wid * G_ROWS_W
    pltpu.sync_copy(idx_hbm.at[wid], idx_v)
    rows = (rows0, rows1)
    gsem = (gs0, gs1)
    wsem = (ws0, ws1)
    prev = None
    wpend = [None, None]
    for c in range(G_NCH):
        b = c & 1
        if wpend[b] is not None:
            wpend[b].wait()
        gh = pltpu.async_copy(x_hbm.at[idx_v.at[c]], rows[b], gsem[b])
        if prev is not None:
            pg, pb, pc = prev
            pg.wait()
            wpend[pb] = pltpu.async_copy(
                rows[pb], out_hbm.at[pl.ds(base + pc * G_CH, G_CH)], wsem[pb])
        prev = (gh, b, c)
    pg, pb, pc = prev
    pg.wait()
    pltpu.sync_copy(rows[pb], out_hbm.at[pl.ds(base + pc * G_CH, G_CH)])
    if wpend[1 - pb] is not None:
        wpend[1 - pb].wait()


def _gather_call(x_packed, idx3):
    mesh = plsc.VectorSubcoreMesh(core_axis_name="c", subcore_axis_name="s")
    f = functools.partial(
        pl.kernel,
        mesh=mesh,
        out_type=jax.ShapeDtypeStruct((PADDED, PACKED), jnp.int32),
        scratch_types=[
            pltpu.VMEM((G_NCH, G_CH), jnp.int32),
            pltpu.VMEM((G_CH, PACKED), jnp.int32),
            pltpu.VMEM((G_CH, PACKED), jnp.int32),
            pltpu.SemaphoreType.DMA,
            pltpu.SemaphoreType.DMA,
            pltpu.SemaphoreType.DMA,
            pltpu.SemaphoreType.DMA,
        ],
    )(_gather_body)
    return f(x_packed, idx3)


def _combine_body(os_hbm, pp_hbm, out_hbm, idx_v, buf_a0, buf_b0, buf_a1,
                  buf_b1, sa0, sb0, sa1, sb1, ws0, ws1):
    wid = lax.axis_index("s") * SC_CORES + lax.axis_index("c")
    base = wid * C_ROWS_W
    pltpu.sync_copy(pp_hbm.at[wid], idx_v)
    bufs = ((buf_a0, buf_b0), (buf_a1, buf_b1))
    sems = ((sa0, sb0), (sa1, sb1))
    wsem = (ws0, ws1)

    def add_into_a(buf_a, buf_b):
        def row_body(i, _):
            for j in range(N_EMBED // 16):
                off = j * 16
                buf_a[i, pl.ds(off, 16)] = (
                    buf_a[i, pl.ds(off, 16)] + buf_b[i, pl.ds(off, 16)])
            return 0
        lax.fori_loop(0, C_CH, row_body, 0)

    prev = None
    wpend = [None, None]
    for c in range(C_NCH):
        b = c & 1
        if wpend[b] is not None:
            wpend[b].wait()
        ga = pltpu.async_copy(os_hbm.at[idx_v.at[c, 0]], bufs[b][0],
                              sems[b][0])
        gb = pltpu.async_copy(os_hbm.at[idx_v.at[c, 1]], bufs[b][1],
                              sems[b][1])
        if prev is not None:
            pga, pgb, pb, pc = prev
            pga.wait()
            pgb.wait()
            add_into_a(bufs[pb][0], bufs[pb][1])
            wpend[pb] = pltpu.async_copy(
                bufs[pb][0], out_hbm.at[pl.ds(base + pc * C_CH, C_CH)],
                wsem[pb])
        prev = (ga, gb, b, c)
    pga, pgb, pb, pc = prev
    pga.wait()
    pgb.wait()
    add_into_a(bufs[pb][0], bufs[pb][1])
    pltpu.sync_copy(bufs[pb][0], out_hbm.at[pl.ds(base + pc * C_CH, C_CH)])
    if wpend[1 - pb] is not None:
        wpend[1 - pb].wait()


def _combine_call(out_sorted, pp):
    mesh = plsc.VectorSubcoreMesh(core_axis_name="c", subcore_axis_name="s")
    f = functools.partial(
        pl.kernel,
        mesh=mesh,
        out_type=jax.ShapeDtypeStruct((N_TOKENS, N_EMBED), jnp.float32),
        scratch_types=[
            pltpu.VMEM((C_NCH, 2, C_CH), jnp.int32),
            pltpu.VMEM((C_CH, N_EMBED), jnp.float32),
            pltpu.VMEM((C_CH, N_EMBED), jnp.float32),
            pltpu.VMEM((C_CH, N_EMBED), jnp.float32),
            pltpu.VMEM((C_CH, N_EMBED), jnp.float32),
            pltpu.SemaphoreType.DMA,
            pltpu.SemaphoreType.DMA,
            pltpu.SemaphoreType.DMA,
            pltpu.SemaphoreType.DMA,
            pltpu.SemaphoreType.DMA,
            pltpu.SemaphoreType.DMA,
        ],
    )(_combine_body)
    return f(out_sorted, pp)


def _routing_metadata(top2i, top2w):
    ef = top2i.reshape(-1)
    oh = (ef[:, None] == jnp.arange(NUM_EXPERTS, dtype=jnp.int32)[None, :])
    cum = jnp.cumsum(oh.astype(jnp.int32), axis=0)
    counts = cum[-1]
    rank = jnp.take_along_axis(cum, ef[:, None], axis=1)[:, 0] - 1
    pc = ((counts + BLK - 1) // BLK) * BLK
    cum_pc = jnp.cumsum(pc)
    po = cum_pc - pc
    dest = (po[ef] + rank).astype(jnp.int32)
    tok = (jnp.arange(N_ASSIGN, dtype=jnp.int32) // TOP_K)
    tok_sorted = jnp.zeros((PADDED,), jnp.int32).at[dest].set(tok)
    ws = jnp.zeros((PADDED,), jnp.float32).at[dest].set(top2w.reshape(-1))
    be = jnp.searchsorted(
        cum_pc, jnp.arange(NUM_BLOCKS, dtype=jnp.int32) * BLK, side='right')
    be = jnp.minimum(be, NUM_EXPERTS - 1).astype(jnp.int32)
    return dest, tok_sorted, ws, be


def kernel(x, gate_W, gate_b, W1, b1, W2, b2):
    b, t, c = x.shape
    x_flat = x.reshape(-1, c)

    gw_pad = jnp.zeros((N_EMBED, GATE_PAD), jnp.float32)
    gw_pad = lax.dynamic_update_slice(gw_pad, gate_W, (0, 0))
    gb_pad = jnp.full((1, GATE_PAD), -jnp.inf, jnp.float32)
    gb_pad = lax.dynamic_update_slice(gb_pad, gate_b[None, :], (0, 0))
    i1, i2, wa, wb = _gating_call(x_flat, gw_pad, gb_pad)
    top2i = jnp.stack([i1[:, 0], i2[:, 0]], axis=1)
    top2w = jnp.stack([wa[:, 0], wb[:, 0]], axis=1)

    dest, tok_sorted, ws, be = _routing_metadata(top2i, top2w)

    x_packed = lax.bitcast_convert_type(
        x_flat.astype(jnp.bfloat16).reshape(N_TOKENS, PACKED, 2), jnp.int32)
    idx3 = tok_sorted.reshape(NW, G_NCH, G_CH)
    xg_packed = _gather_call(x_packed, idx3)
    xg = lax.bitcast_convert_type(
        xg_packed, jnp.bfloat16).reshape(PADDED, N_EMBED)

    h = _ffn1_call(be, xg, W1, b1.reshape(NUM_EXPERTS, 1, HIDDEN))
    out_sorted = _ffn2_call(be, h, W2, b2.reshape(NUM_EXPERTS, 1, N_EMBED),
                            ws[:, None])

    pp = dest.reshape(N_TOKENS, TOP_K).reshape(NW, C_NCH, C_CH, TOP_K)
    pp = jnp.transpose(pp, (0, 1, 3, 2))
    final = _combine_call(out_sorted, pp)

    return final.reshape(b, t, c)


# speedup vs baseline: 1.6209x; 1.6209x over previous
import functools

import jax
import jax.numpy as jnp
from jax import lax
from jax.experimental import pallas as pl
from jax.experimental.pallas import tpu as pltpu
from jax.experimental.pallas import tpu_sc as plsc

N_EMBED = 1024
NUM_EXPERTS = 8
TOP_K = 2
HIDDEN = 4 * N_EMBED
N_TOKENS = 4096
N_ASSIGN = N_TOKENS * TOP_K

BLK = 256
NUM_BLOCKS = N_ASSIGN // BLK + NUM_EXPERTS
PADDED = NUM_BLOCKS * BLK

SC_CORES = 2
SC_SUBCORES = 16
NW = SC_CORES * SC_SUBCORES

G_ROWS_W = PADDED // NW
G_CH = 64
G_NCH = G_ROWS_W // G_CH

C_ROWS_W = N_TOKENS // NW
C_CH = 16
C_NCH = C_ROWS_W // C_CH

GATE_PAD = 128
GATE_ROWS = 512


HALF = N_EMBED // 2


def _pack_bf16_pair(x):
    bits = lax.bitcast_convert_type(x, jnp.int32)
    rb = (bits + 0x7FFF + ((bits >> 16) & 1)) >> 16
    lo = rb[:, :HALF] & 0xFFFF
    hi = rb[:, HALF:] << 16
    return lo | hi


def _unpack_bf16_pair(w):
    lo = lax.bitcast_convert_type(w << 16, jnp.float32)
    hi = lax.bitcast_convert_type(w & jnp.int32(-65536), jnp.float32)
    return lo.astype(jnp.bfloat16), hi.astype(jnp.bfloat16)


def _gating_body(x_ref, gw_ref, gb_ref, i1_ref, i2_ref, wa_ref, wb_ref,
                 xp_ref):
    xp_ref[...] = _pack_bf16_pair(x_ref[...])
    logits = jnp.dot(x_ref[...], gw_ref[...],
                     preferred_element_type=jnp.float32) + gb_ref[...]
    col = lax.broadcasted_iota(jnp.int32, (GATE_ROWS, GATE_PAD), 1)
    m1 = jnp.max(logits, axis=1, keepdims=True)
    i1 = jnp.min(jnp.where(logits == m1, col, GATE_PAD), axis=1, keepdims=True)
    masked = jnp.where(col == i1, -jnp.inf, logits)
    m2 = jnp.max(masked, axis=1, keepdims=True)
    i2 = jnp.min(jnp.where(masked == m2, col, GATE_PAD), axis=1, keepdims=True)
    e2 = jnp.exp(m2 - m1)
    denom = 1.0 + e2
    i1_ref[...] = jnp.broadcast_to(i1, (GATE_ROWS, GATE_PAD))
    i2_ref[...] = jnp.broadcast_to(i2, (GATE_ROWS, GATE_PAD))
    wa_ref[...] = jnp.broadcast_to(1.0 / denom, (GATE_ROWS, GATE_PAD))
    wb_ref[...] = jnp.broadcast_to(e2 / denom, (GATE_ROWS, GATE_PAD))


def _gating_call(x_flat, gw_pad, gb_pad):
    n = x_flat.shape[0]
    grid = (n // GATE_ROWS,)
    out_shape = [
        jax.ShapeDtypeStruct((n, GATE_PAD), jnp.int32),
        jax.ShapeDtypeStruct((n, GATE_PAD), jnp.int32),
        jax.ShapeDtypeStruct((n, GATE_PAD), jnp.float32),
        jax.ShapeDtypeStruct((n, GATE_PAD), jnp.float32),
        jax.ShapeDtypeStruct((n, HALF), jnp.int32),
    ]
    spec_rows = pl.BlockSpec((GATE_ROWS, N_EMBED), lambda g: (g, 0))
    spec_out = pl.BlockSpec((GATE_ROWS, GATE_PAD), lambda g: (g, 0))
    return pl.pallas_call(
        _gating_body,
        grid=grid,
        in_specs=[
            spec_rows,
            pl.BlockSpec((N_EMBED, GATE_PAD), lambda g: (0, 0)),
            pl.BlockSpec((1, GATE_PAD), lambda g: (0, 0)),
        ],
        out_specs=[spec_out, spec_out, spec_out, spec_out,
                   pl.BlockSpec((GATE_ROWS, HALF), lambda g: (g, 0))],
        out_shape=out_shape,
    )(x_flat, gw_pad, gb_pad)


def _ffn1_body(be_ref, xg_ref, w1_ref, b1_ref, h_ref):
    x_lo, x_hi = _unpack_bf16_pair(xg_ref[...])
    w1 = w1_ref[0].astype(jnp.bfloat16)
    h = jnp.dot(x_lo, w1[:HALF], preferred_element_type=jnp.float32)
    h = h + jnp.dot(x_hi, w1[HALF:], preferred_element_type=jnp.float32)
    h_ref[...] = jnp.maximum(h + b1_ref[0], 0.0).astype(jnp.bfloat16)


def _ffn1_call(be, xg, W1, b1):
    grid_spec = pltpu.PrefetchScalarGridSpec(
        num_scalar_prefetch=1,
        grid=(NUM_BLOCKS,),
        in_specs=[
            pl.BlockSpec((BLK, HALF), lambda g, be: (g, 0)),
            pl.BlockSpec((1, N_EMBED, HIDDEN), lambda g, be: (be[g], 0, 0)),
            pl.BlockSpec((1, 1, HIDDEN), lambda g, be: (be[g], 0, 0)),
        ],
        out_specs=pl.BlockSpec((BLK, HIDDEN), lambda g, be: (g, 0)),
    )
    return pl.pallas_call(
        _ffn1_body,
        grid_spec=grid_spec,
        out_shape=jax.ShapeDtypeStruct((PADDED, HIDDEN), jnp.bfloat16),
    )(be, xg, W1, b1)


def _ffn2_body(be_ref, h_ref, w2_ref, b2_ref, ws_ref, out_ref):
    w2 = w2_ref[0].astype(jnp.bfloat16)
    o = jnp.dot(h_ref[...], w2, preferred_element_type=jnp.float32)
    out_ref[...] = (o + b2_ref[0]) * ws_ref[...]


def _ffn2_call(be, h, W2, b2, ws):
    grid_spec = pltpu.PrefetchScalarGridSpec(
        num_scalar_prefetch=1,
        grid=(NUM_BLOCKS,),
        in_specs=[
            pl.BlockSpec((BLK, HIDDEN), lambda g, be: (g, 0)),
            pl.BlockSpec((1, HIDDEN, N_EMBED), lambda g, be: (be[g], 0, 0)),
            pl.BlockSpec((1, 1, N_EMBED), lambda g, be: (be[g], 0, 0)),
            pl.BlockSpec((BLK, 1), lambda g, be: (g, 0)),
        ],
        out_specs=pl.BlockSpec((BLK, N_EMBED), lambda g, be: (g, 0)),
    )
    return pl.pallas_call(
        _ffn2_body,
        grid_spec=grid_spec,
        out_shape=jax.ShapeDtypeStruct((PADDED, N_EMBED), jnp.float32),
    )(be, h, W2, b2, ws)


def _gather_body(x_hbm, idx_hbm, out_hbm, idx_v, rows0, rows1, gs0, gs1,
                 ws0, ws1):
    wid = lax.axis_index("s") * SC_CORES + lax.axis_index("c")
    base = wid * G_ROWS_W
    pltpu.sync_copy(idx_hbm.at[wid], idx_v)
    rows = (rows0, rows1)
    gsem = (gs0, gs1)
    wsem = (ws0, ws1)
    prev = None
    wpend = [None, None]
    for c in range(G_NCH):
        b = c & 1
        if wpend[b] is not None:
            wpend[b].wait()
        gh = pltpu.async_copy(x_hbm.at[idx_v.at[c]], rows[b], gsem[b])
        if prev is not None:
            pg, pb, pc = prev
            pg.wait()
            wpend[pb] = pltpu.async_copy(
                rows[pb], out_hbm.at[pl.ds(base + pc * G_CH, G_CH)], wsem[pb])
        prev = (gh, b, c)
    pg, pb, pc = prev
    pg.wait()
    pltpu.sync_copy(rows[pb], out_hbm.at[pl.ds(base + pc * G_CH, G_CH)])
    if wpend[1 - pb] is not None:
        wpend[1 - pb].wait()


def _gather_call(x_packed, idx3):
    mesh = plsc.VectorSubcoreMesh(core_axis_name="c", subcore_axis_name="s")
    f = functools.partial(
        pl.kernel,
        mesh=mesh,
        out_type=jax.ShapeDtypeStruct((PADDED, HALF), jnp.int32),
        scratch_types=[
            pltpu.VMEM((G_NCH, G_CH), jnp.int32),
            pltpu.VMEM((G_CH, HALF), jnp.int32),
            pltpu.VMEM((G_CH, HALF), jnp.int32),
            pltpu.SemaphoreType.DMA,
            pltpu.SemaphoreType.DMA,
            pltpu.SemaphoreType.DMA,
            pltpu.SemaphoreType.DMA,
        ],
    )(_gather_body)
    return f(x_packed, idx3)


def _combine_body(os_hbm, pp_hbm, out_hbm, idx_v, buf_a0, buf_b0, buf_a1,
                  buf_b1, sa0, sb0, sa1, sb1, ws0, ws1):
    wid = lax.axis_index("s") * SC_CORES + lax.axis_index("c")
    base = wid * C_ROWS_W
    pltpu.sync_copy(pp_hbm.at[wid], idx_v)
    bufs = ((buf_a0, buf_b0), (buf_a1, buf_b1))
    sems = ((sa0, sb0), (sa1, sb1))
    wsem = (ws0, ws1)

    def add_into_a(buf_a, buf_b):
        def row_body(i, _):
            for j in range(N_EMBED // 16):
                off = j * 16
                buf_a[i, pl.ds(off, 16)] = (
                    buf_a[i, pl.ds(off, 16)] + buf_b[i, pl.ds(off, 16)])
            return 0
        lax.fori_loop(0, C_CH, row_body, 0)

    prev = None
    wpend = [None, None]
    for c in range(C_NCH):
        b = c & 1
        if wpend[b] is not None:
            wpend[b].wait()
        ga = pltpu.async_copy(os_hbm.at[idx_v.at[c, 0]], bufs[b][0],
                              sems[b][0])
        gb = pltpu.async_copy(os_hbm.at[idx_v.at[c, 1]], bufs[b][1],
                              sems[b][1])
        if prev is not None:
            pga, pgb, pb, pc = prev
            pga.wait()
            pgb.wait()
            add_into_a(bufs[pb][0], bufs[pb][1])
            wpend[pb] = pltpu.async_copy(
                bufs[pb][0], out_hbm.at[pl.ds(base + pc * C_CH, C_CH)],
                wsem[pb])
        prev = (ga, gb, b, c)
    pga, pgb, pb, pc = prev
    pga.wait()
    pgb.wait()
    add_into_a(bufs[pb][0], bufs[pb][1])
    pltpu.sync_copy(bufs[pb][0], out_hbm.at[pl.ds(base + pc * C_CH, C_CH)])
    if wpend[1 - pb] is not None:
        wpend[1 - pb].wait()


def _combine_call(out_sorted, pp):
    mesh = plsc.VectorSubcoreMesh(core_axis_name="c", subcore_axis_name="s")
    f = functools.partial(
        pl.kernel,
        mesh=mesh,
        out_type=jax.ShapeDtypeStruct((N_TOKENS, N_EMBED), jnp.float32),
        scratch_types=[
            pltpu.VMEM((C_NCH, 2, C_CH), jnp.int32),
            pltpu.VMEM((C_CH, N_EMBED), jnp.float32),
            pltpu.VMEM((C_CH, N_EMBED), jnp.float32),
            pltpu.VMEM((C_CH, N_EMBED), jnp.float32),
            pltpu.VMEM((C_CH, N_EMBED), jnp.float32),
            pltpu.SemaphoreType.DMA,
            pltpu.SemaphoreType.DMA,
            pltpu.SemaphoreType.DMA,
            pltpu.SemaphoreType.DMA,
            pltpu.SemaphoreType.DMA,
            pltpu.SemaphoreType.DMA,
        ],
    )(_combine_body)
    return f(out_sorted, pp)


def _routing_metadata(top2i, top2w):
    ef = top2i.reshape(-1)
    oh = (ef[:, None] == jnp.arange(NUM_EXPERTS, dtype=jnp.int32)[None, :])
    cum = jnp.cumsum(oh.astype(jnp.int32), axis=0)
    counts = cum[-1]
    rank = jnp.take_along_axis(cum, ef[:, None], axis=1)[:, 0] - 1
    pc = ((counts + BLK - 1) // BLK) * BLK
    cum_pc = jnp.cumsum(pc)
    po = cum_pc - pc
    dest = (po[ef] + rank).astype(jnp.int32)
    tok = (jnp.arange(N_ASSIGN, dtype=jnp.int32) // TOP_K)
    tok_sorted = jnp.zeros((PADDED,), jnp.int32).at[dest].set(tok)
    ws = jnp.zeros((PADDED,), jnp.float32).at[dest].set(top2w.reshape(-1))
    be = jnp.searchsorted(
        cum_pc, jnp.arange(NUM_BLOCKS, dtype=jnp.int32) * BLK, side='right')
    be = jnp.minimum(be, NUM_EXPERTS - 1).astype(jnp.int32)
    return dest, tok_sorted, ws, be


def kernel(x, gate_W, gate_b, W1, b1, W2, b2):
    b, t, c = x.shape
    x_flat = x.reshape(-1, c)

    gw_pad = jnp.zeros((N_EMBED, GATE_PAD), jnp.float32)
    gw_pad = lax.dynamic_update_slice(gw_pad, gate_W, (0, 0))
    gb_pad = jnp.full((1, GATE_PAD), -jnp.inf, jnp.float32)
    gb_pad = lax.dynamic_update_slice(gb_pad, gate_b[None, :], (0, 0))
    i1, i2, wa, wb, x_packed = _gating_call(x_flat, gw_pad, gb_pad)
    top2i = jnp.stack([i1[:, 0], i2[:, 0]], axis=1)
    top2w = jnp.stack([wa[:, 0], wb[:, 0]], axis=1)

    dest, tok_sorted, ws, be = _routing_metadata(top2i, top2w)

    idx3 = tok_sorted.reshape(NW, G_NCH, G_CH)
    xg = _gather_call(x_packed, idx3)

    h = _ffn1_call(be, xg, W1, b1.reshape(NUM_EXPERTS, 1, HIDDEN))
    out_sorted = _ffn2_call(be, h, W2, b2.reshape(NUM_EXPERTS, 1, N_EMBED),
                            ws[:, None])

    pp = dest.reshape(N_TOKENS, TOP_K).reshape(NW, C_NCH, C_CH, TOP_K)
    pp = jnp.transpose(pp, (0, 1, 3, 2))
    final = _combine_call(out_sorted, pp)

    return final.reshape(b, t, c)


# speedup vs baseline: 1.6212x; 1.0002x over previous
import functools

import jax
import jax.numpy as jnp
from jax import lax
from jax.experimental import pallas as pl
from jax.experimental.pallas import tpu as pltpu
from jax.experimental.pallas import tpu_sc as plsc

N_EMBED = 1024
NUM_EXPERTS = 8
TOP_K = 2
HIDDEN = 4 * N_EMBED
N_TOKENS = 4096
N_ASSIGN = N_TOKENS * TOP_K

BLK = 256
NUM_BLOCKS = N_ASSIGN // BLK + NUM_EXPERTS
PADDED = NUM_BLOCKS * BLK

SC_CORES = 2
SC_SUBCORES = 16
NW = SC_CORES * SC_SUBCORES

G_ROWS_W = PADDED // NW
G_CH = 32
G_NCH = G_ROWS_W // G_CH
G_NBUF = 4
G_AHEAD = 3

C_ROWS_W = N_TOKENS // NW
C_CH = 16
C_NCH = C_ROWS_W // C_CH

GATE_PAD = 128
GATE_ROWS = 512


HALF = N_EMBED // 2


def _pack_bf16_pair(x):
    bits = lax.bitcast_convert_type(x, jnp.int32)
    rb = (bits + 0x7FFF + ((bits >> 16) & 1)) >> 16
    lo = rb[:, :HALF] & 0xFFFF
    hi = rb[:, HALF:] << 16
    return lo | hi


def _unpack_bf16_pair(w):
    lo = lax.bitcast_convert_type(w << 16, jnp.float32)
    hi = lax.bitcast_convert_type(w & jnp.int32(-65536), jnp.float32)
    return lo.astype(jnp.bfloat16), hi.astype(jnp.bfloat16)


def _gating_body(x_ref, gw_ref, gb_ref, i1_ref, i2_ref, wa_ref, wb_ref,
                 xp_ref):
    xp_ref[...] = _pack_bf16_pair(x_ref[...])
    logits = jnp.dot(x_ref[...], gw_ref[...],
                     preferred_element_type=jnp.float32) + gb_ref[...]
    col = lax.broadcasted_iota(jnp.int32, (GATE_ROWS, GATE_PAD), 1)
    m1 = jnp.max(logits, axis=1, keepdims=True)
    i1 = jnp.min(jnp.where(logits == m1, col, GATE_PAD), axis=1, keepdims=True)
    masked = jnp.where(col == i1, -jnp.inf, logits)
    m2 = jnp.max(masked, axis=1, keepdims=True)
    i2 = jnp.min(jnp.where(masked == m2, col, GATE_PAD), axis=1, keepdims=True)
    e2 = jnp.exp(m2 - m1)
    denom = 1.0 + e2
    i1_ref[...] = jnp.broadcast_to(i1, (GATE_ROWS, GATE_PAD))
    i2_ref[...] = jnp.broadcast_to(i2, (GATE_ROWS, GATE_PAD))
    wa_ref[...] = jnp.broadcast_to(1.0 / denom, (GATE_ROWS, GATE_PAD))
    wb_ref[...] = jnp.broadcast_to(e2 / denom, (GATE_ROWS, GATE_PAD))


def _gating_call(x_flat, gw_pad, gb_pad):
    n = x_flat.shape[0]
    grid = (n // GATE_ROWS,)
    out_shape = [
        jax.ShapeDtypeStruct((n, GATE_PAD), jnp.int32),
        jax.ShapeDtypeStruct((n, GATE_PAD), jnp.int32),
        jax.ShapeDtypeStruct((n, GATE_PAD), jnp.float32),
        jax.ShapeDtypeStruct((n, GATE_PAD), jnp.float32),
        jax.ShapeDtypeStruct((n, HALF), jnp.int32),
    ]
    spec_rows = pl.BlockSpec((GATE_ROWS, N_EMBED), lambda g: (g, 0))
    spec_out = pl.BlockSpec((GATE_ROWS, GATE_PAD), lambda g: (g, 0))
    return pl.pallas_call(
        _gating_body,
        grid=grid,
        in_specs=[
            spec_rows,
            pl.BlockSpec((N_EMBED, GATE_PAD), lambda g: (0, 0)),
            pl.BlockSpec((1, GATE_PAD), lambda g: (0, 0)),
        ],
        out_specs=[spec_out, spec_out, spec_out, spec_out,
                   pl.BlockSpec((GATE_ROWS, HALF), lambda g: (g, 0))],
        out_shape=out_shape,
    )(x_flat, gw_pad, gb_pad)


def _ffn1_body(be_ref, xg_ref, w1_ref, b1_ref, h_ref):
    x_lo, x_hi = _unpack_bf16_pair(xg_ref[...])
    w1 = w1_ref[0].astype(jnp.bfloat16)
    h = jnp.dot(x_lo, w1[:HALF], preferred_element_type=jnp.float32)
    h = h + jnp.dot(x_hi, w1[HALF:], preferred_element_type=jnp.float32)
    h_ref[...] = jnp.maximum(h + b1_ref[0], 0.0).astype(jnp.bfloat16)


def _ffn1_call(be, xg, W1, b1):
    grid_spec = pltpu.PrefetchScalarGridSpec(
        num_scalar_prefetch=1,
        grid=(NUM_BLOCKS,),
        in_specs=[
            pl.BlockSpec((BLK, HALF), lambda g, be: (g, 0)),
            pl.BlockSpec((1, N_EMBED, HIDDEN), lambda g, be: (be[g], 0, 0)),
            pl.BlockSpec((1, 1, HIDDEN), lambda g, be: (be[g], 0, 0)),
        ],
        out_specs=pl.BlockSpec((BLK, HIDDEN), lambda g, be: (g, 0)),
    )
    return pl.pallas_call(
        _ffn1_body,
        grid_spec=grid_spec,
        out_shape=jax.ShapeDtypeStruct((PADDED, HIDDEN), jnp.bfloat16),
    )(be, xg, W1, b1)


def _ffn2_body(be_ref, h_ref, w2_ref, b2_ref, ws_ref, out_ref):
    w2 = w2_ref[0].astype(jnp.bfloat16)
    o = jnp.dot(h_ref[...], w2, preferred_element_type=jnp.float32)
    out_ref[...] = (o + b2_ref[0]) * ws_ref[...]


def _ffn2_call(be, h, W2, b2, ws):
    grid_spec = pltpu.PrefetchScalarGridSpec(
        num_scalar_prefetch=1,
        grid=(NUM_BLOCKS,),
        in_specs=[
            pl.BlockSpec((BLK, HIDDEN), lambda g, be: (g, 0)),
            pl.BlockSpec((1, HIDDEN, N_EMBED), lambda g, be: (be[g], 0, 0)),
            pl.BlockSpec((1, 1, N_EMBED), lambda g, be: (be[g], 0, 0)),
            pl.BlockSpec((BLK, 1), lambda g, be: (g, 0)),
        ],
        out_specs=pl.BlockSpec((BLK, N_EMBED), lambda g, be: (g, 0)),
    )
    return pl.pallas_call(
        _ffn2_body,
        grid_spec=grid_spec,
        out_shape=jax.ShapeDtypeStruct((PADDED, N_EMBED), jnp.float32),
    )(be, h, W2, b2, ws)


def _gather_body(x_hbm, idx_hbm, out_hbm, idx_v, *bufs_and_sems):
    rows = bufs_and_sems[:G_NBUF]
    gsem = bufs_and_sems[G_NBUF:2 * G_NBUF]
    wsem = bufs_and_sems[2 * G_NBUF:3 * G_NBUF]
    wid = lax.axis_index("s") * SC_CORES + lax.axis_index("c")
    base = wid * G_ROWS_W
    pltpu.sync_copy(idx_hbm.at[wid], idx_v)
    gh = [None] * G_NBUF
    wh = [None] * G_NBUF
    for c in range(G_NCH + G_AHEAD):
        if c < G_NCH:
            b = c % G_NBUF
            if wh[b] is not None:
                wh[b].wait()
            gh[b] = pltpu.async_copy(x_hbm.at[idx_v.at[c]], rows[b], gsem[b])
        d = c - G_AHEAD
        if d >= 0:
            b = d % G_NBUF
            gh[b].wait()
            wh[b] = pltpu.async_copy(
                rows[b], out_hbm.at[pl.ds(base + d * G_CH, G_CH)], wsem[b])
    for b in range(G_NBUF):
        if wh[b] is not None:
            wh[b].wait()


def _gather_call(x_packed, idx3):
    mesh = plsc.VectorSubcoreMesh(core_axis_name="c", subcore_axis_name="s")
    f = functools.partial(
        pl.kernel,
        mesh=mesh,
        out_type=jax.ShapeDtypeStruct((PADDED, HALF), jnp.int32),
        scratch_types=(
            [pltpu.VMEM((G_NCH, G_CH), jnp.int32)]
            + [pltpu.VMEM((G_CH, HALF), jnp.int32)] * G_NBUF
            + [pltpu.SemaphoreType.DMA] * (2 * G_NBUF)
        ),
    )(_gather_body)
    return f(x_packed, idx3)


def _combine_body(os_hbm, pp_hbm, out_hbm, idx_v, buf_a0, buf_b0, buf_a1,
                  buf_b1, sa0, sb0, sa1, sb1, ws0, ws1):
    wid = lax.axis_index("s") * SC_CORES + lax.axis_index("c")
    base = wid * C_ROWS_W
    pltpu.sync_copy(pp_hbm.at[wid], idx_v)
    bufs = ((buf_a0, buf_b0), (buf_a1, buf_b1))
    sems = ((sa0, sb0), (sa1, sb1))
    wsem = (ws0, ws1)

    def add_into_a(buf_a, buf_b):
        def row_body(i, _):
            for j in range(N_EMBED // 16):
                off = j * 16
                buf_a[i, pl.ds(off, 16)] = (
                    buf_a[i, pl.ds(off, 16)] + buf_b[i, pl.ds(off, 16)])
            return 0
        lax.fori_loop(0, C_CH, row_body, 0)

    prev = None
    wpend = [None, None]
    for c in range(C_NCH):
        b = c & 1
        if wpend[b] is not None:
            wpend[b].wait()
        ga = pltpu.async_copy(os_hbm.at[idx_v.at[c, 0]], bufs[b][0],
                              sems[b][0])
        gb = pltpu.async_copy(os_hbm.at[idx_v.at[c, 1]], bufs[b][1],
                              sems[b][1])
        if prev is not None:
            pga, pgb, pb, pc = prev
            pga.wait()
            pgb.wait()
            add_into_a(bufs[pb][0], bufs[pb][1])
            wpend[pb] = pltpu.async_copy(
                bufs[pb][0], out_hbm.at[pl.ds(base + pc * C_CH, C_CH)],
                wsem[pb])
        prev = (ga, gb, b, c)
    pga, pgb, pb, pc = prev
    pga.wait()
    pgb.wait()
    add_into_a(bufs[pb][0], bufs[pb][1])
    pltpu.sync_copy(bufs[pb][0], out_hbm.at[pl.ds(base + pc * C_CH, C_CH)])
    if wpend[1 - pb] is not None:
        wpend[1 - pb].wait()


def _combine_call(out_sorted, pp):
    mesh = plsc.VectorSubcoreMesh(core_axis_name="c", subcore_axis_name="s")
    f = functools.partial(
        pl.kernel,
        mesh=mesh,
        out_type=jax.ShapeDtypeStruct((N_TOKENS, N_EMBED), jnp.float32),
        scratch_types=[
            pltpu.VMEM((C_NCH, 2, C_CH), jnp.int32),
            pltpu.VMEM((C_CH, N_EMBED), jnp.float32),
            pltpu.VMEM((C_CH, N_EMBED), jnp.float32),
            pltpu.VMEM((C_CH, N_EMBED), jnp.float32),
            pltpu.VMEM((C_CH, N_EMBED), jnp.float32),
            pltpu.SemaphoreType.DMA,
            pltpu.SemaphoreType.DMA,
            pltpu.SemaphoreType.DMA,
            pltpu.SemaphoreType.DMA,
            pltpu.SemaphoreType.DMA,
            pltpu.SemaphoreType.DMA,
        ],
    )(_combine_body)
    return f(out_sorted, pp)


def _routing_metadata(top2i, top2w):
    ef = top2i.reshape(-1)
    oh = (ef[:, None] == jnp.arange(NUM_EXPERTS, dtype=jnp.int32)[None, :])
    cum = jnp.cumsum(oh.astype(jnp.int32), axis=0)
    counts = cum[-1]
    rank = jnp.take_along_axis(cum, ef[:, None], axis=1)[:, 0] - 1
    pc = ((counts + BLK - 1) // BLK) * BLK
    cum_pc = jnp.cumsum(pc)
    po = cum_pc - pc
    dest = (po[ef] + rank).astype(jnp.int32)
    tok = (jnp.arange(N_ASSIGN, dtype=jnp.int32) // TOP_K)
    tok_sorted = jnp.zeros((PADDED,), jnp.int32).at[dest].set(tok)
    ws = jnp.zeros((PADDED,), jnp.float32).at[dest].set(top2w.reshape(-1))
    be = jnp.searchsorted(
        cum_pc, jnp.arange(NUM_BLOCKS, dtype=jnp.int32) * BLK, side='right')
    be = jnp.minimum(be, NUM_EXPERTS - 1).astype(jnp.int32)
    return dest, tok_sorted, ws, be


def kernel(x, gate_W, gate_b, W1, b1, W2, b2):
    b, t, c = x.shape
    x_flat = x.reshape(-1, c)

    gw_pad = jnp.zeros((N_EMBED, GATE_PAD), jnp.float32)
    gw_pad = lax.dynamic_update_slice(gw_pad, gate_W, (0, 0))
    gb_pad = jnp.full((1, GATE_PAD), -jnp.inf, jnp.float32)
    gb_pad = lax.dynamic_update_slice(gb_pad, gate_b[None, :], (0, 0))
    i1, i2, wa, wb, x_packed = _gating_call(x_flat, gw_pad, gb_pad)
    top2i = jnp.stack([i1[:, 0], i2[:, 0]], axis=1)
    top2w = jnp.stack([wa[:, 0], wb[:, 0]], axis=1)

    dest, tok_sorted, ws, be = _routing_metadata(top2i, top2w)

    idx3 = tok_sorted.reshape(NW, G_NCH, G_CH)
    xg = _gather_call(x_packed, idx3)

    h = _ffn1_call(be, xg, W1, b1.reshape(NUM_EXPERTS, 1, HIDDEN))
    out_sorted = _ffn2_call(be, h, W2, b2.reshape(NUM_EXPERTS, 1, N_EMBED),
                            ws[:, None])

    pp = dest.reshape(N_TOKENS, TOP_K).reshape(NW, C_NCH, C_CH, TOP_K)
    pp = jnp.transpose(pp, (0, 1, 3, 2))
    final = _combine_call(out_sorted, pp)

    return final.reshape(b, t, c)


# speedup vs baseline: 1.6785x; 1.0354x over previous
import functools

import jax
import jax.numpy as jnp
from jax import lax
from jax.experimental import pallas as pl
from jax.experimental.pallas import tpu as pltpu
from jax.experimental.pallas import tpu_sc as plsc

N_EMBED = 1024
NUM_EXPERTS = 8
TOP_K = 2
HIDDEN = 4 * N_EMBED
N_TOKENS = 4096
N_ASSIGN = N_TOKENS * TOP_K

BLK = 256
NUM_BLOCKS = N_ASSIGN // BLK + NUM_EXPERTS
PADDED = NUM_BLOCKS * BLK

SC_CORES = 2
SC_SUBCORES = 16
NW = SC_CORES * SC_SUBCORES

G_ROWS_W = PADDED // NW
G_CH = 32
G_NCH = G_ROWS_W // G_CH
G_NBUF = 4
G_AHEAD = 3

C_ROWS_W = N_TOKENS // NW
C_CH = 16
C_NCH = C_ROWS_W // C_CH

GATE_PAD = 128
GATE_ROWS = 512


HALF = N_EMBED // 2


def _pack_bf16_pair(x):
    bits = lax.bitcast_convert_type(x, jnp.int32)
    rb = (bits + 0x7FFF + ((bits >> 16) & 1)) >> 16
    lo = rb[:, :HALF] & 0xFFFF
    hi = rb[:, HALF:] << 16
    return lo | hi


def _unpack_bf16_pair(w):
    lo = lax.bitcast_convert_type(w << 16, jnp.float32)
    hi = lax.bitcast_convert_type(w & jnp.int32(-65536), jnp.float32)
    return lo.astype(jnp.bfloat16), hi.astype(jnp.bfloat16)


def _gating_body(x_ref, gw_ref, gb_ref, i1_ref, i2_ref, wa_ref, wb_ref,
                 xp_ref):
    xp_ref[...] = _pack_bf16_pair(x_ref[...])
    logits = jnp.dot(x_ref[...], gw_ref[...],
                     preferred_element_type=jnp.float32) + gb_ref[...]
    col = lax.broadcasted_iota(jnp.int32, (GATE_ROWS, GATE_PAD), 1)
    m1 = jnp.max(logits, axis=1, keepdims=True)
    i1 = jnp.min(jnp.where(logits == m1, col, GATE_PAD), axis=1, keepdims=True)
    masked = jnp.where(col == i1, -jnp.inf, logits)
    m2 = jnp.max(masked, axis=1, keepdims=True)
    i2 = jnp.min(jnp.where(masked == m2, col, GATE_PAD), axis=1, keepdims=True)
    e2 = jnp.exp(m2 - m1)
    denom = 1.0 + e2
    i1_ref[...] = jnp.broadcast_to(i1, (GATE_ROWS, GATE_PAD))
    i2_ref[...] = jnp.broadcast_to(i2, (GATE_ROWS, GATE_PAD))
    wa_ref[...] = jnp.broadcast_to(1.0 / denom, (GATE_ROWS, GATE_PAD))
    wb_ref[...] = jnp.broadcast_to(e2 / denom, (GATE_ROWS, GATE_PAD))


def _gating_call(x_flat, gw_pad, gb_pad):
    n = x_flat.shape[0]
    grid = (n // GATE_ROWS,)
    out_shape = [
        jax.ShapeDtypeStruct((n, GATE_PAD), jnp.int32),
        jax.ShapeDtypeStruct((n, GATE_PAD), jnp.int32),
        jax.ShapeDtypeStruct((n, GATE_PAD), jnp.float32),
        jax.ShapeDtypeStruct((n, GATE_PAD), jnp.float32),
        jax.ShapeDtypeStruct((n, HALF), jnp.int32),
    ]
    spec_rows = pl.BlockSpec((GATE_ROWS, N_EMBED), lambda g: (g, 0))
    spec_out = pl.BlockSpec((GATE_ROWS, GATE_PAD), lambda g: (g, 0))
    return pl.pallas_call(
        _gating_body,
        grid=grid,
        in_specs=[
            spec_rows,
            pl.BlockSpec((N_EMBED, GATE_PAD), lambda g: (0, 0)),
            pl.BlockSpec((1, GATE_PAD), lambda g: (0, 0)),
        ],
        out_specs=[spec_out, spec_out, spec_out, spec_out,
                   pl.BlockSpec((GATE_ROWS, HALF), lambda g: (g, 0))],
        out_shape=out_shape,
    )(x_flat, gw_pad, gb_pad)


def _ffn1_body(be_ref, xg_ref, w1_ref, b1_ref, h_ref):
    x_lo, x_hi = _unpack_bf16_pair(xg_ref[...])
    w1 = w1_ref[0].astype(jnp.bfloat16)
    h = jnp.dot(x_lo, w1[:HALF], preferred_element_type=jnp.float32)
    h = h + jnp.dot(x_hi, w1[HALF:], preferred_element_type=jnp.float32)
    h_ref[...] = jnp.maximum(h + b1_ref[0], 0.0).astype(jnp.bfloat16)


def _ffn1_call(be, xg, W1, b1):
    grid_spec = pltpu.PrefetchScalarGridSpec(
        num_scalar_prefetch=1,
        grid=(NUM_BLOCKS,),
        in_specs=[
            pl.BlockSpec((BLK, HALF), lambda g, be: (g, 0)),
            pl.BlockSpec((1, N_EMBED, HIDDEN), lambda g, be: (be[g], 0, 0)),
            pl.BlockSpec((1, 1, HIDDEN), lambda g, be: (be[g], 0, 0)),
        ],
        out_specs=pl.BlockSpec((BLK, HIDDEN), lambda g, be: (g, 0)),
    )
    return pl.pallas_call(
        _ffn1_body,
        grid_spec=grid_spec,
        out_shape=jax.ShapeDtypeStruct((PADDED, HIDDEN), jnp.bfloat16),
    )(be, xg, W1, b1)


def _ffn2_body(be_ref, h_ref, w2_ref, b2_ref, ws_ref, out_ref):
    w2 = w2_ref[0].astype(jnp.bfloat16)
    o = jnp.dot(h_ref[...], w2, preferred_element_type=jnp.float32)
    out_ref[...] = (o + b2_ref[0]) * ws_ref[...]


def _ffn2_call(be, h, W2, b2, ws):
    grid_spec = pltpu.PrefetchScalarGridSpec(
        num_scalar_prefetch=1,
        grid=(NUM_BLOCKS,),
        in_specs=[
            pl.BlockSpec((BLK, HIDDEN), lambda g, be: (g, 0)),
            pl.BlockSpec((1, HIDDEN, N_EMBED), lambda g, be: (be[g], 0, 0)),
            pl.BlockSpec((1, 1, N_EMBED), lambda g, be: (be[g], 0, 0)),
            pl.BlockSpec((BLK, 1), lambda g, be: (g, 0)),
        ],
        out_specs=pl.BlockSpec((BLK, N_EMBED), lambda g, be: (g, 0)),
    )
    return pl.pallas_call(
        _ffn2_body,
        grid_spec=grid_spec,
        out_shape=jax.ShapeDtypeStruct((PADDED, N_EMBED), jnp.float32),
    )(be, h, W2, b2, ws)


def _gather_body(x_hbm, idx_hbm, out_hbm, idx_v, *bufs_and_sems):
    rows = bufs_and_sems[:G_NBUF]
    gsem = bufs_and_sems[G_NBUF:2 * G_NBUF]
    wsem = bufs_and_sems[2 * G_NBUF:3 * G_NBUF]
    wid = lax.axis_index("s") * SC_CORES + lax.axis_index("c")
    base = wid * G_ROWS_W
    pltpu.sync_copy(idx_hbm.at[wid], idx_v)
    gh = [None] * G_NBUF
    wh = [None] * G_NBUF
    for c in range(G_NCH + G_AHEAD):
        if c < G_NCH:
            b = c % G_NBUF
            if wh[b] is not None:
                wh[b].wait()
            gh[b] = pltpu.async_copy(x_hbm.at[idx_v.at[c]], rows[b], gsem[b])
        d = c - G_AHEAD
        if d >= 0:
            b = d % G_NBUF
            gh[b].wait()
            wh[b] = pltpu.async_copy(
                rows[b], out_hbm.at[pl.ds(base + d * G_CH, G_CH)], wsem[b])
    for b in range(G_NBUF):
        if wh[b] is not None:
            wh[b].wait()


def _gather_call(x_packed, idx3):
    mesh = plsc.VectorSubcoreMesh(core_axis_name="c", subcore_axis_name="s")
    f = functools.partial(
        pl.kernel,
        mesh=mesh,
        out_type=jax.ShapeDtypeStruct((PADDED, HALF), jnp.int32),
        scratch_types=(
            [pltpu.VMEM((G_NCH, G_CH), jnp.int32)]
            + [pltpu.VMEM((G_CH, HALF), jnp.int32)] * G_NBUF
            + [pltpu.SemaphoreType.DMA] * (2 * G_NBUF)
        ),
    )(_gather_body)
    return f(x_packed, idx3)


def _combine_body(os_hbm, pp_hbm, out_hbm, idx_v, buf_a0, buf_b0, buf_a1,
                  buf_b1, sa0, sb0, sa1, sb1, ws0, ws1):
    wid = lax.axis_index("s") * SC_CORES + lax.axis_index("c")
    base = wid * C_ROWS_W
    pltpu.sync_copy(pp_hbm.at[wid], idx_v)
    bufs = ((buf_a0, buf_b0), (buf_a1, buf_b1))
    sems = ((sa0, sb0), (sa1, sb1))
    wsem = (ws0, ws1)

    def add_into_a(buf_a, buf_b):
        def row_body(i, _):
            for j in range(N_EMBED // 16):
                off = j * 16
                buf_a[i, pl.ds(off, 16)] = (
                    buf_a[i, pl.ds(off, 16)] + buf_b[i, pl.ds(off, 16)])
            return 0
        lax.fori_loop(0, C_CH, row_body, 0)

    prev = None
    wpend = [None, None]
    for c in range(C_NCH):
        b = c & 1
        if wpend[b] is not None:
            wpend[b].wait()
        ga = pltpu.async_copy(os_hbm.at[idx_v.at[c, 0]], bufs[b][0],
                              sems[b][0])
        gb = pltpu.async_copy(os_hbm.at[idx_v.at[c, 1]], bufs[b][1],
                              sems[b][1])
        if prev is not None:
            pga, pgb, pb, pc = prev
            pga.wait()
            pgb.wait()
            add_into_a(bufs[pb][0], bufs[pb][1])
            wpend[pb] = pltpu.async_copy(
                bufs[pb][0], out_hbm.at[pl.ds(base + pc * C_CH, C_CH)],
                wsem[pb])
        prev = (ga, gb, b, c)
    pga, pgb, pb, pc = prev
    pga.wait()
    pgb.wait()
    add_into_a(bufs[pb][0], bufs[pb][1])
    pltpu.sync_copy(bufs[pb][0], out_hbm.at[pl.ds(base + pc * C_CH, C_CH)])
    if wpend[1 - pb] is not None:
        wpend[1 - pb].wait()


def _combine_call(out_sorted, pp):
    mesh = plsc.VectorSubcoreMesh(core_axis_name="c", subcore_axis_name="s")
    f = functools.partial(
        pl.kernel,
        mesh=mesh,
        out_type=jax.ShapeDtypeStruct((N_TOKENS, N_EMBED), jnp.float32),
        scratch_types=[
            pltpu.VMEM((C_NCH, 2, C_CH), jnp.int32),
            pltpu.VMEM((C_CH, N_EMBED), jnp.float32),
            pltpu.VMEM((C_CH, N_EMBED), jnp.float32),
            pltpu.VMEM((C_CH, N_EMBED), jnp.float32),
            pltpu.VMEM((C_CH, N_EMBED), jnp.float32),
            pltpu.SemaphoreType.DMA,
            pltpu.SemaphoreType.DMA,
            pltpu.SemaphoreType.DMA,
            pltpu.SemaphoreType.DMA,
            pltpu.SemaphoreType.DMA,
            pltpu.SemaphoreType.DMA,
        ],
    )(_combine_body)
    return f(out_sorted, pp)


def _routing_metadata(top2i, top2w):
    ef = top2i.reshape(-1)
    ohf = (ef.reshape(64, 128)[:, :, None]
           == jnp.arange(NUM_EXPERTS, dtype=jnp.int32)).astype(jnp.float32)
    lt = (jnp.arange(128)[:, None] >= jnp.arange(128)[None, :]
          ).astype(jnp.float32)
    cs = jnp.einsum('ij,sje->sie', lt, ohf)
    seg = cs[:, -1, :]
    ltx = (jnp.arange(64)[:, None] > jnp.arange(64)[None, :]
           ).astype(jnp.float32)
    cumf = cs + (ltx @ seg)[:, None, :]
    counts = seg.sum(axis=0).astype(jnp.int32)
    rank = ((cumf * ohf).sum(-1) - 1.0).reshape(N_ASSIGN).astype(jnp.int32)
    pc = ((counts + BLK - 1) // BLK) * BLK
    cum_pc = jnp.cumsum(pc)
    po = cum_pc - pc
    oh2 = (ef[:, None] == jnp.arange(NUM_EXPERTS, dtype=jnp.int32)[None, :])
    dest = (rank + (oh2 * po[None, :]).sum(axis=1)).astype(jnp.int32)
    tok = (jnp.arange(N_ASSIGN, dtype=jnp.int32) // TOP_K)
    ts_ws = jnp.zeros((PADDED, 2), jnp.float32).at[dest].set(
        jnp.stack([tok.astype(jnp.float32), top2w.reshape(-1)], axis=1))
    tok_sorted = ts_ws[:, 0].astype(jnp.int32)
    ws = ts_ws[:, 1]
    be = jnp.searchsorted(
        cum_pc, jnp.arange(NUM_BLOCKS, dtype=jnp.int32) * BLK, side='right')
    be = jnp.minimum(be, NUM_EXPERTS - 1).astype(jnp.int32)
    return dest, tok_sorted, ws, be


def kernel(x, gate_W, gate_b, W1, b1, W2, b2):
    b, t, c = x.shape
    x_flat = x.reshape(-1, c)

    gw_pad = jnp.zeros((N_EMBED, GATE_PAD), jnp.float32)
    gw_pad = lax.dynamic_update_slice(gw_pad, gate_W, (0, 0))
    gb_pad = jnp.full((1, GATE_PAD), -jnp.inf, jnp.float32)
    gb_pad = lax.dynamic_update_slice(gb_pad, gate_b[None, :], (0, 0))
    i1, i2, wa, wb, x_packed = _gating_call(x_flat, gw_pad, gb_pad)
    top2i = jnp.stack([i1[:, 0], i2[:, 0]], axis=1)
    top2w = jnp.stack([wa[:, 0], wb[:, 0]], axis=1)

    dest, tok_sorted, ws, be = _routing_metadata(top2i, top2w)

    idx3 = tok_sorted.reshape(NW, G_NCH, G_CH)
    xg = _gather_call(x_packed, idx3)

    h = _ffn1_call(be, xg, W1, b1.reshape(NUM_EXPERTS, 1, HIDDEN))
    out_sorted = _ffn2_call(be, h, W2, b2.reshape(NUM_EXPERTS, 1, N_EMBED),
                            ws[:, None])

    pp = dest.reshape(N_TOKENS, TOP_K).reshape(NW, C_NCH, C_CH, TOP_K)
    pp = jnp.transpose(pp, (0, 1, 3, 2))
    final = _combine_call(out_sorted, pp)

    return final.reshape(b, t, c)
